# Initial kernel scaffold; baseline (speedup 1.0000x reference)
import jax
import jax.numpy as jnp
from jax.experimental import pallas as pl


def kernel(x, W1, b1, W2, b2, We, be):
    raise NotImplementedError("write your pallas kernel here")


# speedup vs baseline: 8.7925x; 8.7925x over previous
import functools

import jax
import jax.numpy as jnp
from jax import lax
from jax.experimental import pallas as pl
from jax.experimental.pallas import tpu as pltpu
from jax.experimental.pallas import tpu_sc as plsc

B, W, C, S = 32, 2048, 128, 32
TR = W // 2
U = W - TR
_SQRT2 = 1.4142135623730951
_RCHUNK = 256
_GCHUNK = 128
_NGC = U // _GCHUNK


def _gelu(v):
    return v * 0.5 * (1.0 + lax.erf(v / _SQRT2))


def _softmax(v):
    m = jnp.max(v, axis=-1, keepdims=True)
    e = jnp.exp(v - m)
    return e / jnp.sum(e, axis=-1, keepdims=True)


def _win32(a):
    for k in (1, 2, 4, 8, 16):
        z = jnp.zeros((k, a.shape[1]), a.dtype)
        a = a + jnp.concatenate([z, a[: a.shape[0] - k]], axis=0)
    return a


def _select_kernel(x_ref, idx_ref):
    b = pl.program_id(0)
    xb = x_ref[0]
    xq = xb.astype(jnp.bfloat16).astype(jnp.float32)
    x2q = (xb * xb).astype(jnp.bfloat16).astype(jnp.float32)
    sx = _win32(xq)
    sxx = _win32(x2q)
    wpos = lax.broadcasted_iota(jnp.int32, (W, 1), 0).astype(jnp.float32)
    den = jnp.minimum(wpos + 1.0, float(S))
    ltrm = sx / den
    ltrd = sxx / den - ltrm * ltrm
    num = jnp.sum(ltrd, axis=1, keepdims=True)
    dnm = jnp.sum(ltrm, axis=1, keepdims=True)
    score_c = num / dnm
    score_r = jnp.transpose(score_c)

    rank_chunks = []
    for cb in range(W // _RCHUNK):
        s_i = score_c[cb * _RCHUNK:(cb + 1) * _RCHUNK]
        i_i = (lax.broadcasted_iota(jnp.int32, (_RCHUNK, W), 0).astype(jnp.float32)
               + float(cb * _RCHUNK))
        j_j = lax.broadcasted_iota(jnp.int32, (_RCHUNK, W), 1).astype(jnp.float32)
        lt = score_r < s_i
        tie = (score_r == s_i) & (j_j < i_i)
        cmp = jnp.where(lt | tie, 1.0, 0.0)
        rank_chunks.append(jnp.sum(cmp, axis=1, keepdims=True))
    rank_c = jnp.concatenate(rank_chunks, axis=0)
    rank_r = jnp.transpose(rank_c)

    boff = (b * W).astype(jnp.float32)
    for pb in range(U // _RCHUNK):
        p_i = (lax.broadcasted_iota(jnp.int32, (_RCHUNK, W), 0).astype(jnp.float32)
               + float(pb * _RCHUNK))
        j_j = lax.broadcasted_iota(jnp.int32, (_RCHUNK, W), 1).astype(jnp.float32)
        sel = jnp.where(rank_r == p_i, j_j, 0.0)
        inv = jnp.sum(sel, axis=1, keepdims=True)
        idx_ref[0, 0, pb * _RCHUNK:(pb + 1) * _RCHUNK] = (
            jnp.transpose(inv + boff).reshape(_RCHUNK))


def _encode_kernel(x_ref, u_ref, We_ref, be_ref, W1_ref, b1_ref, W2_ref, b2_ref,
                   att_ref, rec_ref, out_ref, cg_ref):
    xb = x_ref[0]
    ub = u_ref[0]
    We = We_ref[...]
    be = be_ref[...]

    out_ref[0] = _gelu(jax.lax.dot_general(
        xb, We, (((1,), (0,)), ((), ())), preferred_element_type=jnp.float32) + be)
    gx = jax.lax.dot_general(xb, xb, (((0,), (0,)), ((), ())),
                             preferred_element_type=jnp.float32)
    cg_ref[0] = _softmax(gx / float(W))

    ux = _gelu(jax.lax.dot_general(
        ub, We, (((1,), (0,)), ((), ())), preferred_element_type=jnp.float32) + be)
    gu = jax.lax.dot_general(ub, ub, (((0,), (0,)), ((), ())),
                             preferred_element_type=jnp.float32)
    att_ref[0] = _softmax(gu / float(U))

    h = _gelu(jax.lax.dot_general(
        ux, W1_ref[...], (((1,), (0,)), ((), ())), preferred_element_type=jnp.float32)
        + b1_ref[...])
    z = jax.lax.dot_general(
        h, W2_ref[...], (((1,), (0,)), ((), ())), preferred_element_type=jnp.float32) \
        + b2_ref[...]
    rec_ref[0] = 1.0 / (1.0 + jnp.exp(-z))


def _sc_gather(xflat, gidx):
    mesh = plsc.VectorSubcoreMesh(core_axis_name="c", subcore_axis_name="s")

    @functools.partial(
        pl.kernel,
        mesh=mesh,
        out_type=jax.ShapeDtypeStruct((B * U, C), jnp.float32),
        scratch_types=[
            pltpu.VMEM((_NGC, _GCHUNK), jnp.int32),
            pltpu.VMEM((_GCHUNK, C), jnp.float32),
            pltpu.SemaphoreType.DMA,
        ],
    )
    def gather_k(x_hbm, idx_hbm, out_hbm, idx_v, rows_v, sem):
        wid = lax.axis_index("s") * 2 + lax.axis_index("c")
        pltpu.sync_copy(idx_hbm.at[pl.ds(wid * _NGC, _NGC)], idx_v)
        for j in range(_NGC):
            pltpu.async_copy(x_hbm.at[idx_v.at[j]], rows_v, sem).wait()
            pltpu.sync_copy(rows_v, out_hbm.at[pl.ds(wid * U + j * _GCHUNK, _GCHUNK)])

    return gather_k(xflat, gidx)


def kernel(x, W1, b1, W2, b2, We, be):
    idxf = pl.pallas_call(
        _select_kernel,
        grid=(B,),
        in_specs=[pl.BlockSpec((1, W, C), lambda b: (b, 0, 0))],
        out_specs=pl.BlockSpec((1, 1, U), lambda b: (b, 0, 0)),
        out_shape=jax.ShapeDtypeStruct((B, 1, U), jnp.float32),
    )(x)

    gidx = idxf.astype(jnp.int32).reshape(B * _NGC, _GCHUNK)
    unm = _sc_gather(x.reshape(B * W, C), gidx).reshape(B, U, C)

    be2 = be.reshape(1, C)
    b12 = b1.reshape(1, C)
    b22 = b2.reshape(1, C)
    full = lambda i, j: (lambda b: (0, 0))
    att, rec, out, cg = pl.pallas_call(
        _encode_kernel,
        grid=(B,),
        in_specs=[
            pl.BlockSpec((1, W, C), lambda b: (b, 0, 0)),
            pl.BlockSpec((1, U, C), lambda b: (b, 0, 0)),
            pl.BlockSpec((C, C), lambda b: (0, 0)),
            pl.BlockSpec((1, C), lambda b: (0, 0)),
            pl.BlockSpec((C, C), lambda b: (0, 0)),
            pl.BlockSpec((1, C), lambda b: (0, 0)),
            pl.BlockSpec((C, C), lambda b: (0, 0)),
            pl.BlockSpec((1, C), lambda b: (0, 0)),
        ],
        out_specs=[
            pl.BlockSpec((1, C, C), lambda b: (b, 0, 0)),
            pl.BlockSpec((1, U, C), lambda b: (b, 0, 0)),
            pl.BlockSpec((1, W, C), lambda b: (b, 0, 0)),
            pl.BlockSpec((1, C, C), lambda b: (b, 0, 0)),
        ],
        out_shape=[
            jax.ShapeDtypeStruct((B, C, C), jnp.float32),
            jax.ShapeDtypeStruct((B, U, C), jnp.float32),
            jax.ShapeDtypeStruct((B, W, C), jnp.float32),
            jax.ShapeDtypeStruct((B, C, C), jnp.float32),
        ],
    )(x, unm, We, be2, W1, b12, W2, b22)
    return (att, rec, out, cg)


# speedup vs baseline: 11.8972x; 1.3531x over previous
import functools

import jax
import jax.numpy as jnp
from jax import lax
from jax.experimental import pallas as pl
from jax.experimental.pallas import tpu as pltpu
from jax.experimental.pallas import tpu_sc as plsc

B, W, C, S = 32, 2048, 128, 32
TR = W // 2
U = W - TR
_SQRT2 = 1.4142135623730951
_RC = 256
_NB = W // _RC
_GCHUNK = 128
_NGC = U // _GCHUNK


def _gelu(v):
    return v * 0.5 * (1.0 + lax.erf(v / _SQRT2))


def _softmax(v):
    m = jnp.max(v, axis=-1, keepdims=True)
    e = jnp.exp(v - m)
    return e / jnp.sum(e, axis=-1, keepdims=True)


def _win32(a):
    for k in (1, 2, 4, 8, 16):
        z = jnp.zeros((k, a.shape[1]), a.dtype)
        a = a + jnp.concatenate([z, a[: a.shape[0] - k]], axis=0)
    return a


def _div_den(a):
    wpos = lax.broadcasted_iota(jnp.int32, (S, 1), 0).astype(jnp.float32)
    den = jnp.minimum(wpos + 1.0, float(S))
    return jnp.concatenate([a[:S] / den, a[S:] * (1.0 / S)], axis=0)


def _select_kernel(x_ref, rank_ref):
    xb = x_ref[0]
    xq = xb.astype(jnp.bfloat16).astype(jnp.float32)
    x2q = (xb * xb).astype(jnp.bfloat16).astype(jnp.float32)
    ltrm = _div_den(_win32(xq))
    ltr2 = _div_den(_win32(x2q))
    ltrd = ltr2 - ltrm * ltrm
    num = jnp.sum(ltrd, axis=1, keepdims=True)
    dnm = jnp.sum(ltrm, axis=1, keepdims=True)
    score_c = num / dnm
    score_r = jnp.transpose(score_c)

    rank_cols = []
    for cb in range(_NB):
        s_i = score_c[cb * _RC:(cb + 1) * _RC]
        acc = jnp.zeros((_RC, 1), jnp.float32)
        for jc in range(_NB):
            s_j = score_r[:, jc * _RC:(jc + 1) * _RC]
            if jc < cb:
                cmp = (s_j <= s_i).astype(jnp.float32)
            elif jc > cb:
                cmp = (s_j < s_i).astype(jnp.float32)
            else:
                ii = lax.broadcasted_iota(jnp.int32, (_RC, _RC), 0)
                jj = lax.broadcasted_iota(jnp.int32, (_RC, _RC), 1)
                cmp = ((s_j < s_i).astype(jnp.float32)
                       + (s_j == s_i).astype(jnp.float32)
                       * (jj < ii).astype(jnp.float32))
            acc = acc + jnp.sum(cmp, axis=1, keepdims=True)
        rank_cols.append(acc)
    rank_c = jnp.concatenate(rank_cols, axis=0)
    boff = (pl.program_id(0) * W).astype(jnp.float32)
    rank_ref[0, 0, :] = jnp.transpose(rank_c + boff).reshape(W).astype(jnp.int32)


def _encode_x_kernel(x_ref, We_ref, be_ref, out_ref, cg_ref):
    xb = x_ref[0]
    out_ref[0] = _gelu(lax.dot_general(
        xb, We_ref[...], (((1,), (0,)), ((), ())),
        preferred_element_type=jnp.float32) + be_ref[...])
    gx = lax.dot_general(xb, xb, (((0,), (0,)), ((), ())),
                         preferred_element_type=jnp.float32)
    cg_ref[0] = _softmax(gx / float(W))


def _encode_u_kernel(u_ref, We_ref, be_ref, W1_ref, b1_ref, W2_ref, b2_ref,
                     att_ref, rec_ref):
    ub = u_ref[0]
    ux = _gelu(lax.dot_general(
        ub, We_ref[...], (((1,), (0,)), ((), ())),
        preferred_element_type=jnp.float32) + be_ref[...])
    gu = lax.dot_general(ub, ub, (((0,), (0,)), ((), ())),
                         preferred_element_type=jnp.float32)
    att_ref[0] = _softmax(gu / float(U))
    h = _gelu(lax.dot_general(
        ux, W1_ref[...], (((1,), (0,)), ((), ())),
        preferred_element_type=jnp.float32) + b1_ref[...])
    z = lax.dot_general(
        h, W2_ref[...], (((1,), (0,)), ((), ())),
        preferred_element_type=jnp.float32) + b2_ref[...]
    rec_ref[0] = 1.0 / (1.0 + jnp.exp(-z))


def _sc_scatter(xflat, grank):
    mesh = plsc.VectorSubcoreMesh(core_axis_name="c", subcore_axis_name="s")
    NCH = W // _GCHUNK

    @functools.partial(
        pl.kernel,
        mesh=mesh,
        out_type=jax.ShapeDtypeStruct((B * W, C), jnp.float32),
        scratch_types=[
            pltpu.VMEM((NCH, _GCHUNK), jnp.int32),
            pltpu.VMEM((2, _GCHUNK, C), jnp.float32),
            pltpu.SemaphoreType.DMA,
            pltpu.SemaphoreType.DMA,
            pltpu.SemaphoreType.DMA,
            pltpu.SemaphoreType.DMA,
        ],
    )
    def scat_k(x_hbm, grank_hbm, out_hbm, ridx_v, rows_v, l0, l1, s0, s1):
        wid = lax.axis_index("s") * 2 + lax.axis_index("c")
        pltpu.sync_copy(grank_hbm.at[pl.ds(wid * NCH, NCH)], ridx_v)
        lsem = (l0, l1)
        ssem = (s0, s1)
        loads = [None, None]
        stores = [None, None]
        loads[0] = pltpu.async_copy(
            x_hbm.at[pl.ds(wid * W, _GCHUNK)], rows_v.at[0], lsem[0])
        for k in range(NCH):
            cur = k % 2
            nxt = (k + 1) % 2
            if k + 1 < NCH:
                if stores[nxt] is not None:
                    stores[nxt].wait()
                loads[nxt] = pltpu.async_copy(
                    x_hbm.at[pl.ds(wid * W + (k + 1) * _GCHUNK, _GCHUNK)],
                    rows_v.at[nxt], lsem[nxt])
            loads[cur].wait()
            stores[cur] = pltpu.async_copy(
                rows_v.at[cur], out_hbm.at[ridx_v.at[k]], ssem[cur])
        stores[0].wait()
        stores[1].wait()

    return scat_k(xflat, grank)


def kernel(x, W1, b1, W2, b2, We, be):
    rank = pl.pallas_call(
        _select_kernel,
        grid=(B,),
        in_specs=[pl.BlockSpec((1, W, C), lambda b: (b, 0, 0))],
        out_specs=pl.BlockSpec((1, 1, W), lambda b: (b, 0, 0)),
        out_shape=jax.ShapeDtypeStruct((B, 1, W), jnp.int32),
    )(x)

    scat = _sc_scatter(x.reshape(B * W, C),
                       rank.reshape(B * (W // _GCHUNK), _GCHUNK))
    unm = scat.reshape(B, W, C)

    be2 = be.reshape(1, C)
    out, cg = pl.pallas_call(
        _encode_x_kernel,
        grid=(B,),
        in_specs=[
            pl.BlockSpec((1, W, C), lambda b: (b, 0, 0)),
            pl.BlockSpec((C, C), lambda b: (0, 0)),
            pl.BlockSpec((1, C), lambda b: (0, 0)),
        ],
        out_specs=[
            pl.BlockSpec((1, W, C), lambda b: (b, 0, 0)),
            pl.BlockSpec((1, C, C), lambda b: (b, 0, 0)),
        ],
        out_shape=[
            jax.ShapeDtypeStruct((B, W, C), jnp.float32),
            jax.ShapeDtypeStruct((B, C, C), jnp.float32),
        ],
    )(x, We, be2)

    att, rec = pl.pallas_call(
        _encode_u_kernel,
        grid=(B,),
        in_specs=[
            pl.BlockSpec((1, U, C), lambda b: (b, 0, 0)),
            pl.BlockSpec((C, C), lambda b: (0, 0)),
            pl.BlockSpec((1, C), lambda b: (0, 0)),
            pl.BlockSpec((C, C), lambda b: (0, 0)),
            pl.BlockSpec((1, C), lambda b: (0, 0)),
            pl.BlockSpec((C, C), lambda b: (0, 0)),
            pl.BlockSpec((1, C), lambda b: (0, 0)),
        ],
        out_specs=[
            pl.BlockSpec((1, C, C), lambda b: (b, 0, 0)),
            pl.BlockSpec((1, U, C), lambda b: (b, 0, 0)),
        ],
        out_shape=[
            jax.ShapeDtypeStruct((B, C, C), jnp.float32),
            jax.ShapeDtypeStruct((B, U, C), jnp.float32),
        ],
    )(unm, We, be2, W1, b1.reshape(1, C), W2, b2.reshape(1, C))
    return (att, rec, out, cg)


# speedup vs baseline: 14.1915x; 1.1929x over previous
import functools

import jax
import jax.numpy as jnp
from jax import lax
from jax.experimental import pallas as pl
from jax.experimental.pallas import tpu as pltpu
from jax.experimental.pallas import tpu_sc as plsc

B, W, C, S = 32, 2048, 128, 32
TR = W // 2
U = W - TR
_SQRT2 = 1.4142135623730951
_RC = 256
_NB = W // _RC
_GCHUNK = 128
_NGC = U // _GCHUNK


def _gelu(v):
    return v * 0.5 * (1.0 + lax.erf(v / _SQRT2))


def _softmax(v):
    m = jnp.max(v, axis=-1, keepdims=True)
    e = jnp.exp(v - m)
    return e / jnp.sum(e, axis=-1, keepdims=True)


def _win32(a):
    for k in (1, 2, 4, 8, 16):
        z = jnp.zeros((k, a.shape[1]), a.dtype)
        a = a + jnp.concatenate([z, a[: a.shape[0] - k]], axis=0)
    return a


def _div_den(a):
    wpos = lax.broadcasted_iota(jnp.int32, (S, 1), 0).astype(jnp.float32)
    den = jnp.minimum(wpos + 1.0, float(S))
    return jnp.concatenate([a[:S] / den, a[S:] * (1.0 / S)], axis=0)


def _select_x_kernel(x_ref, We_ref, be_ref, rank_ref, out_ref, cg_ref):
    xb = x_ref[0]

    out_ref[0] = _gelu(lax.dot_general(
        xb, We_ref[...], (((1,), (0,)), ((), ())),
        preferred_element_type=jnp.float32) + be_ref[...])
    gx = lax.dot_general(xb, xb, (((0,), (0,)), ((), ())),
                         preferred_element_type=jnp.float32)
    cg_ref[0] = _softmax(gx / float(W))

    xq = xb.astype(jnp.bfloat16).astype(jnp.float32)
    x2q = (xb * xb).astype(jnp.bfloat16).astype(jnp.float32)
    ltrm = _div_den(_win32(xq))
    ltr2 = _div_den(_win32(x2q))
    ltrd = ltr2 - ltrm * ltrm
    num = jnp.sum(ltrd, axis=1, keepdims=True)
    dnm = jnp.sum(ltrm, axis=1, keepdims=True)
    score_c = num / dnm
    score_r = jnp.transpose(score_c)

    ii = lax.broadcasted_iota(jnp.int32, (_RC, _RC), 0)
    jj = lax.broadcasted_iota(jnp.int32, (_RC, _RC), 1)
    tri = (jj < ii).astype(jnp.float32)
    rank_cols = []
    for cb in range(_NB):
        s_i = score_c[cb * _RC:(cb + 1) * _RC]
        acc = jnp.zeros((_RC, _RC), jnp.float32)
        for jc in range(_NB):
            s_j = score_r[:, jc * _RC:(jc + 1) * _RC]
            if jc < cb:
                acc = acc + (s_j <= s_i).astype(jnp.float32)
            elif jc > cb:
                acc = acc + (s_j < s_i).astype(jnp.float32)
            else:
                acc = acc + ((s_j < s_i).astype(jnp.float32)
                             + (s_j == s_i).astype(jnp.float32) * tri)
        rank_cols.append(jnp.sum(acc, axis=1, keepdims=True))
    rank_c = jnp.concatenate(rank_cols, axis=0)
    boff = (pl.program_id(0) * W).astype(jnp.float32)
    rank_ref[0, 0, :] = jnp.transpose(rank_c + boff).reshape(W).astype(jnp.int32)


def _encode_u_kernel(u_ref, We_ref, be_ref, W1_ref, b1_ref, W2_ref, b2_ref,
                     att_ref, rec_ref):
    ub = u_ref[0]
    ux = _gelu(lax.dot_general(
        ub, We_ref[...], (((1,), (0,)), ((), ())),
        preferred_element_type=jnp.float32) + be_ref[...])
    gu = lax.dot_general(ub, ub, (((0,), (0,)), ((), ())),
                         preferred_element_type=jnp.float32)
    att_ref[0] = _softmax(gu / float(U))
    h = _gelu(lax.dot_general(
        ux, W1_ref[...], (((1,), (0,)), ((), ())),
        preferred_element_type=jnp.float32) + b1_ref[...])
    z = lax.dot_general(
        h, W2_ref[...], (((1,), (0,)), ((), ())),
        preferred_element_type=jnp.float32) + b2_ref[...]
    rec_ref[0] = 1.0 / (1.0 + jnp.exp(-z))


def _sc_scatter(xflat, grank):
    mesh = plsc.VectorSubcoreMesh(core_axis_name="c", subcore_axis_name="s")
    NCH = W // _GCHUNK

    @functools.partial(
        pl.kernel,
        mesh=mesh,
        out_type=jax.ShapeDtypeStruct((B * W, C), jnp.float32),
        scratch_types=[
            pltpu.VMEM((NCH, _GCHUNK), jnp.int32),
            pltpu.VMEM((2, _GCHUNK, C), jnp.float32),
            pltpu.SemaphoreType.DMA,
            pltpu.SemaphoreType.DMA,
            pltpu.SemaphoreType.DMA,
            pltpu.SemaphoreType.DMA,
        ],
    )
    def scat_k(x_hbm, grank_hbm, out_hbm, ridx_v, rows_v, l0, l1, s0, s1):
        wid = lax.axis_index("s") * 2 + lax.axis_index("c")
        pltpu.sync_copy(grank_hbm.at[pl.ds(wid * NCH, NCH)], ridx_v)
        lsem = (l0, l1)
        ssem = (s0, s1)
        loads = [None, None]
        stores = [None, None]
        loads[0] = pltpu.async_copy(
            x_hbm.at[pl.ds(wid * W, _GCHUNK)], rows_v.at[0], lsem[0])
        for k in range(NCH):
            cur = k % 2
            nxt = (k + 1) % 2
            if k + 1 < NCH:
                if stores[nxt] is not None:
                    stores[nxt].wait()
                loads[nxt] = pltpu.async_copy(
                    x_hbm.at[pl.ds(wid * W + (k + 1) * _GCHUNK, _GCHUNK)],
                    rows_v.at[nxt], lsem[nxt])
            loads[cur].wait()
            stores[cur] = pltpu.async_copy(
                rows_v.at[cur], out_hbm.at[ridx_v.at[k]], ssem[cur])
        stores[0].wait()
        stores[1].wait()

    return scat_k(xflat, grank)


def kernel(x, W1, b1, W2, b2, We, be):
    be2 = be.reshape(1, C)
    rank, out, cg = pl.pallas_call(
        _select_x_kernel,
        grid=(B,),
        in_specs=[
            pl.BlockSpec((1, W, C), lambda b: (b, 0, 0)),
            pl.BlockSpec((C, C), lambda b: (0, 0)),
            pl.BlockSpec((1, C), lambda b: (0, 0)),
        ],
        out_specs=[
            pl.BlockSpec((1, 1, W), lambda b: (b, 0, 0)),
            pl.BlockSpec((1, W, C), lambda b: (b, 0, 0)),
            pl.BlockSpec((1, C, C), lambda b: (b, 0, 0)),
        ],
        out_shape=[
            jax.ShapeDtypeStruct((B, 1, W), jnp.int32),
            jax.ShapeDtypeStruct((B, W, C), jnp.float32),
            jax.ShapeDtypeStruct((B, C, C), jnp.float32),
        ],
    )(x, We, be2)

    scat = _sc_scatter(x.reshape(B * W, C),
                       rank.reshape(B * (W // _GCHUNK), _GCHUNK))
    unm = scat.reshape(B, W, C)

    att, rec = pl.pallas_call(
        _encode_u_kernel,
        grid=(B,),
        in_specs=[
            pl.BlockSpec((1, U, C), lambda b: (b, 0, 0)),
            pl.BlockSpec((C, C), lambda b: (0, 0)),
            pl.BlockSpec((1, C), lambda b: (0, 0)),
            pl.BlockSpec((C, C), lambda b: (0, 0)),
            pl.BlockSpec((1, C), lambda b: (0, 0)),
            pl.BlockSpec((C, C), lambda b: (0, 0)),
            pl.BlockSpec((1, C), lambda b: (0, 0)),
        ],
        out_specs=[
            pl.BlockSpec((1, C, C), lambda b: (b, 0, 0)),
            pl.BlockSpec((1, U, C), lambda b: (b, 0, 0)),
        ],
        out_shape=[
            jax.ShapeDtypeStruct((B, C, C), jnp.float32),
            jax.ShapeDtypeStruct((B, U, C), jnp.float32),
        ],
    )(unm, We, be2, W1, b1.reshape(1, C), W2, b2.reshape(1, C))
    return (att, rec, out, cg)
